# baseline (speedup 1.0000x reference)
import functools

import jax
import jax.numpy as jnp
import numpy as np
from jax import lax
from jax.experimental import pallas as pl
from jax.experimental.pallas import tpu as pltpu

N_SAMPLES = 1000
NUM_CLASSES = 10
B = 1024

_KA = (1832780943, 270669613)
_KB = (64467757, 2916123636)

_ROT = (13, 15, 26, 6, 17, 29, 16, 24)
_TINY = np.float32(np.finfo(np.float32).tiny)

_S_TILE = 40
_ROWS = NUM_CLASSES * _S_TILE
_STEPS = N_SAMPLES // _S_TILE


def _tf_bits_np(k0, k1, x1):
    M = np.uint32(0xFFFFFFFF)

    def rotl(x, r):
        return (x << np.uint32(r)) | (x >> np.uint32(32 - r))

    ks = (np.uint32(k0), np.uint32(k1), np.uint32(k0 ^ k1 ^ 0x1BD11BDA))
    x0 = np.full_like(x1, ks[0])
    x1 = (x1 + ks[1]).astype(np.uint32)
    for i in range(5):
        for r in _ROT[(i % 2) * 4:(i % 2) * 4 + 4]:
            x0 = (x0 + x1).astype(np.uint32)
            x1 = rotl(x1, r)
            x1 ^= x0
        x0 = (x0 + ks[(i + 1) % 3]).astype(np.uint32)
        x1 = (x1 + np.uint32((int(ks[(i + 2) % 3]) + i + 1) & 0xFFFFFFFF)).astype(np.uint32)
    return x0 ^ x1


def _e_from_bits(bits):
    fb = ((bits >> np.uint32(9)) | np.uint32(0x3F800000)).view(np.float32) \
        - np.float32(1.0)
    u = np.maximum(_TINY, fb * (np.float32(1.0) - _TINY) + _TINY)
    return -np.log(u)


_E_CACHE = None


def _e_table():
    global _E_CACHE
    if _E_CACHE is None:
        r = np.arange(_ROWS, dtype=np.uint32)
        s_off = (r % _S_TILE)[:, None].astype(np.uint32)
        c = (r // _S_TILE)[:, None].astype(np.uint32)
        b = np.arange(B, dtype=np.uint32)[None, :]
        base = s_off * np.uint32(B * NUM_CLASSES) + b * np.uint32(NUM_CLASSES) + c
        steps = (np.arange(_STEPS, dtype=np.uint32)
                 * np.uint32(_S_TILE * B * NUM_CLASSES))[:, None, None]
        ctr = (steps + base[None]).astype(np.uint32)
        ea = _e_from_bits(_tf_bits_np(*_KA, ctr))
        eb = _e_from_bits(_tf_bits_np(*_KB, ctr))
        _E_CACHE = np.concatenate([ea, eb], axis=1)
    return _E_CACHE


def _class_min(q):
    m = q[0:_S_TILE]
    idx = jnp.zeros((_S_TILE, B), jnp.int32)
    for c in range(1, NUM_CLASSES):
        qc = q[c * _S_TILE:(c + 1) * _S_TILE]
        lt = qc < m
        m = jnp.where(lt, qc, m)
        idx = jnp.where(lt, c, idx)
    return m, idx


def _expand80(x):
    return jnp.broadcast_to(x[:, None, :], (NUM_CLASSES, _S_TILE, B)).reshape(_ROWS, B)


def _sampler_kernel(at_ref, bt_ref, y_ref, e_ref, out_ref, ca_ref, t_ref,
                    ra_ref, rb_ref, y8_ref, ci_ref):
    i = pl.program_id(0)

    @pl.when(i == 0)
    def _init():
        ca_ref[...] = jnp.zeros((_ROWS, B), jnp.float32)
        t_ref[...] = jnp.zeros((_S_TILE, B), jnp.float32)
        ra_ref[...] = _expand80(np.float32(1.0) / (at_ref[...] + np.float32(1e-12)))
        rb_ref[...] = _expand80(np.float32(1.0) / (bt_ref[...] + np.float32(1e-12)))
        y8_ref[...] = jnp.broadcast_to(y_ref[...], (_S_TILE, B))
        ci_ref[...] = lax.broadcasted_iota(
            jnp.int32, (NUM_CLASSES, _S_TILE, B), 0).reshape(_ROWS, B)

    e = e_ref[0]
    qa = e[:_ROWS] * ra_ref[...]
    qb = e[_ROWS:] * rb_ref[...]
    _, ia = _class_min(qa)
    _, ib = _class_min(qb)
    mask = (ia >= ib) & (ib == y8_ref[...])
    m80 = jnp.tile(jnp.where(mask, ia, -1), (NUM_CLASSES, 1))
    ca_ref[...] += jnp.where(m80 == ci_ref[...], np.float32(1.0), np.float32(0.0))
    t_ref[...] += mask.astype(jnp.float32)

    @pl.when(i == _STEPS - 1)
    def _finalize():
        at = at_ref[...]
        bt = bt_ref[...]
        ca = ca_ref[...]
        counts_a = ca.reshape(NUM_CLASSES, _S_TILE, B).sum(axis=1)
        total = t_ref[...].sum(axis=0, keepdims=True)
        safe = jnp.maximum(total, np.float32(1.0))
        has = total > np.float32(0.0)
        a_pred = jnp.where(has, counts_a / safe, np.float32(0.0))
        cidx10 = lax.broadcasted_iota(jnp.int32, (NUM_CLASSES, B), 0)
        b_pred = jnp.where(has & (cidx10 == y_ref[...]), total / safe, np.float32(0.0))
        da = at - a_pred
        db = bt - b_pred
        sq = jnp.sum(da * da + db * db, axis=0, keepdims=True)
        out_ref[...] = jnp.sum(sq, axis=1, keepdims=True) \
            / np.float32(2 * B * NUM_CLASSES)


def _sample_loss(a_distrs, b_distrs, y):
    at = a_distrs.T
    bt = b_distrs.T
    y2 = y.reshape(1, B)
    et = jnp.asarray(_e_table())
    out = pl.pallas_call(
        _sampler_kernel,
        grid=(_STEPS,),
        in_specs=[
            pl.BlockSpec((NUM_CLASSES, B), lambda i: (0, 0)),
            pl.BlockSpec((NUM_CLASSES, B), lambda i: (0, 0)),
            pl.BlockSpec((1, B), lambda i: (0, 0)),
            pl.BlockSpec((1, 2 * _ROWS, B), lambda i: (i, 0, 0)),
        ],
        out_specs=pl.BlockSpec((1, 1), lambda i: (0, 0)),
        out_shape=jax.ShapeDtypeStruct((1, 1), jnp.float32),
        scratch_shapes=[
            pltpu.VMEM((_ROWS, B), jnp.float32),
            pltpu.VMEM((_S_TILE, B), jnp.float32),
            pltpu.VMEM((_ROWS, B), jnp.float32),
            pltpu.VMEM((_ROWS, B), jnp.float32),
            pltpu.VMEM((_S_TILE, B), jnp.int32),
            pltpu.VMEM((_ROWS, B), jnp.int32),
        ],
    )(at, bt, y2, et)
    return out[0, 0]


_NB = 256
_NBLK = 2 * B // _NB


def _cnn_kernel(x_ref, w1_ref, b1_ref, w2_ref, b2_ref,
                fw1_ref, fb1_ref, fw2_ref, fb2_ref, out_ref):
    xm = tuple(x_ref[:, r] for r in range(4))
    blocks = []
    for p in range(2):
        for pr in range(2):
            parts = []
            for di in range(5):
                o2 = 2 * pr + p + di
                parts.append(xm[o2 % 4][:, o2 // 4:o2 // 4 + 6, :])
            blocks.append(jnp.concatenate(parts, axis=-1))
    p1in = jnp.concatenate(blocks, axis=1)
    c1 = jnp.dot(p1in.reshape(_NB * 24, 140), w1_ref[...],
                 preferred_element_type=jnp.float32)
    c1 = (c1 + b1_ref[...]).reshape(_NB, 24, 768)
    pj = jnp.maximum(c1[:, :, :384], c1[:, :, 384:])
    p1 = jnp.maximum(pj[:, 0:12, :], pj[:, 12:24, :])
    p2blocks = []
    for p2_ in range(2):
        parts = []
        for di in range(5):
            off = p2_ + di
            base = (off % 2) * 6 + off // 2
            parts.append(p1[:, base:base + 4, :])
        p2blocks.append(jnp.concatenate(parts, axis=-1))
    p2in = jnp.concatenate(p2blocks, axis=1)
    c2 = jnp.dot(p2in.reshape(_NB * 8, 1920), w2_ref[...],
                 preferred_element_type=jnp.float32)
    c2 = (c2 + b2_ref[...]).reshape(_NB, 8, 512)
    qj = jnp.maximum(c2[:, :, :256], c2[:, :, 256:])
    p2 = jnp.maximum(qj[:, 0:4, :], qj[:, 4:8, :])
    h = fb1_ref[...]
    for i in range(4):
        h = h + jnp.dot(p2[:, i, :], fw1_ref[i],
                        preferred_element_type=jnp.float32)
    h = jnp.maximum(h, np.float32(0.0))
    z = jnp.dot(h, fw2_ref[...], preferred_element_type=jnp.float32) + fb2_ref[...]
    m = jnp.max(z, axis=-1, keepdims=True)
    e = jnp.exp(z - m)
    out_ref[...] = e / jnp.sum(e, axis=-1, keepdims=True)


def _sel(njj, nj):
    jj = np.arange(njj)[:, None, None]
    j = np.arange(nj)[None, :, None]
    dj = np.arange(5)[None, None, :]
    return (jj == j + dj).astype(np.float32)


_S1 = _sel(28, 24)
_S2 = _sel(12, 8)


def _band_w1(conv1_w):
    w = jnp.einsum('JjD,oiD->iJjo', jnp.asarray(_S1), conv1_w[:, 0])
    return w.reshape(140, 12, 2, 32).transpose(0, 2, 1, 3).reshape(140, 768)


def _band_w2(conv2_w):
    w = jnp.einsum('JjD,ociD->iJcjo', jnp.asarray(_S2), conv2_w)
    w = w.reshape(1920, 8, 64)
    return w.reshape(1920, 4, 2, 64).transpose(0, 2, 1, 3).reshape(1920, 512)


_PERM = None


def _perm():
    global _PERM
    if _PERM is None:
        kp = np.arange(1024)
        i, j, o = kp // 256, (kp % 256) // 64, kp % 64
        _PERM = o * 16 + i * 4 + j
    return _PERM


def _mnist_net_pallas(imgs, conv1_w, conv1_b, conv2_w, conv2_b,
                      fc1_w, fc1_b, fc2_w, fc2_b):
    xq = imgs.reshape(2 * B, 7, 4, 28).transpose(0, 2, 1, 3)
    w1 = _band_w1(conv1_w)
    b1 = jnp.tile(conv1_b, 24).reshape(1, 768)
    w2 = _band_w2(conv2_w)
    b2 = jnp.tile(conv2_b, 8).reshape(1, 512)
    fw1 = (fc1_w.T.reshape(64, 4, 4, 1024).transpose(1, 2, 0, 3)
           .reshape(4, 256, 1024))
    fb1 = fc1_b.reshape(1, 1024)
    fw2 = jnp.concatenate([fc2_w.T, jnp.zeros((1024, 6), jnp.float32)], axis=1)
    fb2 = jnp.concatenate([fc2_b, jnp.full((6,), -1e30, jnp.float32)]).reshape(1, 16)
    cst = lambda *shape: pl.BlockSpec(shape, lambda i: (0,) * len(shape))
    out = pl.pallas_call(
        _cnn_kernel,
        grid=(_NBLK,),
        in_specs=[
            pl.BlockSpec((_NB, 4, 7, 28), lambda i: (i, 0, 0, 0)),
            cst(140, 768), cst(1, 768),
            cst(1920, 512), cst(1, 512),
            cst(4, 256, 1024), cst(1, 1024),
            cst(1024, 16), cst(1, 16),
        ],
        out_specs=pl.BlockSpec((_NB, 16), lambda i: (i, 0)),
        out_shape=jax.ShapeDtypeStruct((2 * B, 16), jnp.float32),
    )(xq, w1, b1, w2, b2, fw1, fb1, fw2, fb2)
    return out[:, :NUM_CLASSES]


def kernel(a_imgs, b_imgs, y, conv1_w, conv1_b, conv2_w, conv2_b, fc1_w, fc1_b, fc2_w, fc2_b):
    imgs = jnp.concatenate([a_imgs, b_imgs], axis=0)
    distrs = _mnist_net_pallas(imgs, conv1_w, conv1_b, conv2_w, conv2_b,
                               fc1_w, fc1_b, fc2_w, fc2_b)
    a_distrs, b_distrs = distrs[:B], distrs[B:]
    return _sample_loss(a_distrs, b_distrs, y)


# speedup vs baseline: 1.0560x; 1.0560x over previous
import jax
import jax.numpy as jnp
import numpy as np
from jax import lax
from jax.experimental import pallas as pl
from jax.experimental.pallas import tpu as pltpu

N_SAMPLES = 1000
NUM_CLASSES = 10
B = 1024

_KA = (1832780943, 270669613)
_KB = (64467757, 2916123636)

_ROT = (13, 15, 26, 6, 17, 29, 16, 24)
_TINY = np.float32(np.finfo(np.float32).tiny)

_S_TILE = 40
_ROWS = NUM_CLASSES * _S_TILE
_STEPS = N_SAMPLES // _S_TILE


def _tf_bits_np(k0, k1, x1):
    M = np.uint32(0xFFFFFFFF)

    def rotl(x, r):
        return (x << np.uint32(r)) | (x >> np.uint32(32 - r))

    ks = (np.uint32(k0), np.uint32(k1), np.uint32(k0 ^ k1 ^ 0x1BD11BDA))
    x0 = np.full_like(x1, ks[0])
    x1 = (x1 + ks[1]).astype(np.uint32)
    for i in range(5):
        for r in _ROT[(i % 2) * 4:(i % 2) * 4 + 4]:
            x0 = (x0 + x1).astype(np.uint32)
            x1 = rotl(x1, r)
            x1 ^= x0
        x0 = (x0 + ks[(i + 1) % 3]).astype(np.uint32)
        x1 = (x1 + np.uint32((int(ks[(i + 2) % 3]) + i + 1) & 0xFFFFFFFF)).astype(np.uint32)
    return x0 ^ x1


def _e_from_bits(bits):
    fb = ((bits >> np.uint32(9)) | np.uint32(0x3F800000)).view(np.float32) \
        - np.float32(1.0)
    u = np.maximum(_TINY, fb * (np.float32(1.0) - _TINY) + _TINY)
    return -np.log(u)


_E_CACHE = None


def _e_table():
    global _E_CACHE
    if _E_CACHE is None:
        r = np.arange(_ROWS, dtype=np.uint32)
        s_off = (r % _S_TILE)[:, None].astype(np.uint32)
        c = (r // _S_TILE)[:, None].astype(np.uint32)
        b = np.arange(B, dtype=np.uint32)[None, :]
        base = s_off * np.uint32(B * NUM_CLASSES) + b * np.uint32(NUM_CLASSES) + c
        steps = (np.arange(_STEPS, dtype=np.uint32)
                 * np.uint32(_S_TILE * B * NUM_CLASSES))[:, None, None]
        ctr = (steps + base[None]).astype(np.uint32)
        ea = _e_from_bits(_tf_bits_np(*_KA, ctr))
        eb = _e_from_bits(_tf_bits_np(*_KB, ctr))
        _E_CACHE = np.concatenate([ea, eb], axis=1)
    return _E_CACHE


def _class_min(q):
    m = q[0:_S_TILE]
    idx = jnp.zeros((_S_TILE, B), jnp.int32)
    for c in range(1, NUM_CLASSES):
        qc = q[c * _S_TILE:(c + 1) * _S_TILE]
        lt = qc < m
        m = jnp.where(lt, qc, m)
        idx = jnp.where(lt, c, idx)
    return m, idx


def _expand80(x):
    return jnp.broadcast_to(x[:, None, :], (NUM_CLASSES, _S_TILE, B)).reshape(_ROWS, B)


def _sampler_kernel(at_ref, bt_ref, y_ref, e_ref, out_ref, ca_ref, t_ref,
                    ra_ref, rb_ref, y8_ref, ci_ref):
    i = pl.program_id(0)

    @pl.when(i == 0)
    def _init():
        ca_ref[...] = jnp.zeros((_ROWS, B), jnp.float32)
        t_ref[...] = jnp.zeros((_S_TILE, B), jnp.float32)
        ra_ref[...] = _expand80(np.float32(1.0) / (at_ref[...] + np.float32(1e-12)))
        rb_ref[...] = _expand80(np.float32(1.0) / (bt_ref[...] + np.float32(1e-12)))
        y8_ref[...] = jnp.broadcast_to(y_ref[...], (_S_TILE, B))
        ci_ref[...] = lax.broadcasted_iota(
            jnp.int32, (NUM_CLASSES, _S_TILE, B), 0).reshape(_ROWS, B)

    e = e_ref[0]
    qa = e[:_ROWS] * ra_ref[...]
    qb = e[_ROWS:] * rb_ref[...]
    _, ia = _class_min(qa)
    _, ib = _class_min(qb)
    mask = (ia >= ib) & (ib == y8_ref[...])
    m80 = jnp.tile(jnp.where(mask, ia, -1), (NUM_CLASSES, 1))
    ca_ref[...] += jnp.where(m80 == ci_ref[...], np.float32(1.0), np.float32(0.0))
    t_ref[...] += mask.astype(jnp.float32)

    @pl.when(i == _STEPS - 1)
    def _finalize():
        at = at_ref[...]
        bt = bt_ref[...]
        ca = ca_ref[...]
        counts_a = ca.reshape(NUM_CLASSES, _S_TILE, B).sum(axis=1)
        total = t_ref[...].sum(axis=0, keepdims=True)
        safe = jnp.maximum(total, np.float32(1.0))
        has = total > np.float32(0.0)
        a_pred = jnp.where(has, counts_a / safe, np.float32(0.0))
        cidx10 = lax.broadcasted_iota(jnp.int32, (NUM_CLASSES, B), 0)
        b_pred = jnp.where(has & (cidx10 == y_ref[...]), total / safe, np.float32(0.0))
        da = at - a_pred
        db = bt - b_pred
        sq = jnp.sum(da * da + db * db, axis=0, keepdims=True)
        out_ref[...] = jnp.sum(sq, axis=1, keepdims=True) \
            / np.float32(2 * B * NUM_CLASSES)


def _sample_loss(a_distrs, b_distrs, y):
    at = a_distrs.T
    bt = b_distrs.T
    y2 = y.reshape(1, B)
    et = jnp.asarray(_e_table())
    out = pl.pallas_call(
        _sampler_kernel,
        grid=(_STEPS,),
        in_specs=[
            pl.BlockSpec((NUM_CLASSES, B), lambda i: (0, 0)),
            pl.BlockSpec((NUM_CLASSES, B), lambda i: (0, 0)),
            pl.BlockSpec((1, B), lambda i: (0, 0)),
            pl.BlockSpec((1, 2 * _ROWS, B), lambda i: (i, 0, 0)),
        ],
        out_specs=pl.BlockSpec((1, 1), lambda i: (0, 0)),
        out_shape=jax.ShapeDtypeStruct((1, 1), jnp.float32),
        scratch_shapes=[
            pltpu.VMEM((_ROWS, B), jnp.float32),
            pltpu.VMEM((_S_TILE, B), jnp.float32),
            pltpu.VMEM((_ROWS, B), jnp.float32),
            pltpu.VMEM((_ROWS, B), jnp.float32),
            pltpu.VMEM((_S_TILE, B), jnp.int32),
            pltpu.VMEM((_ROWS, B), jnp.int32),
        ],
    )(at, bt, y2, et)
    return out[0, 0]


_NB = 256
_NBLK = 2 * B // _NB


def _cnn_kernel(x_ref, w1_ref, b1_ref, w2_ref, b2_ref,
                fw1_ref, fb1_ref, fw2_ref, fb2_ref, out_ref):
    xm = tuple(x_ref[:, r] for r in range(4))
    blocks = []
    for p in range(2):
        for pr in range(2):
            parts = []
            for di in range(5):
                o2 = 2 * pr + p + di
                parts.append(xm[o2 % 4][:, o2 // 4:o2 // 4 + 6, :])
            blocks.append(jnp.concatenate(parts, axis=-1))
    p1in = jnp.concatenate(blocks, axis=1)
    c1 = jnp.dot(p1in.reshape(_NB * 24, 140).astype(jnp.bfloat16), w1_ref[...],
                 preferred_element_type=jnp.float32)
    c1 = (c1 + b1_ref[...]).reshape(_NB, 24, 768)
    pj = jnp.maximum(c1[:, :, :384], c1[:, :, 384:])
    p1 = jnp.maximum(pj[:, 0:12, :], pj[:, 12:24, :])
    p2blocks = []
    for p2_ in range(2):
        parts = []
        for di in range(5):
            off = p2_ + di
            base = (off % 2) * 6 + off // 2
            parts.append(p1[:, base:base + 4, :])
        p2blocks.append(jnp.concatenate(parts, axis=-1))
    p2in = jnp.concatenate(p2blocks, axis=1)
    c2 = jnp.dot(p2in.reshape(_NB * 8, 1920).astype(jnp.bfloat16), w2_ref[...],
                 preferred_element_type=jnp.float32)
    c2 = (c2 + b2_ref[...]).reshape(_NB, 8, 512)
    qj = jnp.maximum(c2[:, :, :256], c2[:, :, 256:])
    p2 = jnp.maximum(qj[:, 0:4, :], qj[:, 4:8, :])
    h = fb1_ref[...]
    for i in range(4):
        h = h + jnp.dot(p2[:, i, :].astype(jnp.bfloat16), fw1_ref[i],
                        preferred_element_type=jnp.float32)
    h = jnp.maximum(h, np.float32(0.0))
    z = jnp.dot(h, fw2_ref[...], preferred_element_type=jnp.float32) + fb2_ref[...]
    m = jnp.max(z, axis=-1, keepdims=True)
    e = jnp.exp(z - m)
    out_ref[...] = e / jnp.sum(e, axis=-1, keepdims=True)


def _sel(njj, nj):
    jj = np.arange(njj)[:, None, None]
    j = np.arange(nj)[None, :, None]
    dj = np.arange(5)[None, None, :]
    return (jj == j + dj).astype(np.float32)


_S1 = _sel(28, 24)
_S2 = _sel(12, 8)


def _band_w1(conv1_w):
    w = jnp.einsum('JjD,oiD->iJjo', jnp.asarray(_S1), conv1_w[:, 0])
    return w.reshape(140, 12, 2, 32).transpose(0, 2, 1, 3).reshape(140, 768)


def _band_w2(conv2_w):
    w = jnp.einsum('JjD,ociD->iJcjo', jnp.asarray(_S2), conv2_w)
    w = w.reshape(1920, 8, 64)
    return w.reshape(1920, 4, 2, 64).transpose(0, 2, 1, 3).reshape(1920, 512)


def _mnist_net_pallas(imgs, conv1_w, conv1_b, conv2_w, conv2_b,
                      fc1_w, fc1_b, fc2_w, fc2_b):
    xq = imgs.reshape(2 * B, 7, 4, 28).transpose(0, 2, 1, 3)
    w1 = _band_w1(conv1_w).astype(jnp.bfloat16)
    b1 = jnp.tile(conv1_b, 24).reshape(1, 768)
    w2 = _band_w2(conv2_w).astype(jnp.bfloat16)
    b2 = jnp.tile(conv2_b, 8).reshape(1, 512)
    fw1 = (fc1_w.T.reshape(64, 4, 4, 1024).transpose(1, 2, 0, 3)
           .reshape(4, 256, 1024).astype(jnp.bfloat16))
    fb1 = fc1_b.reshape(1, 1024)
    fw2 = jnp.concatenate([fc2_w.T, jnp.zeros((1024, 6), jnp.float32)], axis=1)
    fb2 = jnp.concatenate([fc2_b, jnp.full((6,), -1e30, jnp.float32)]).reshape(1, 16)
    cst = lambda *shape: pl.BlockSpec(shape, lambda i: (0,) * len(shape))
    out = pl.pallas_call(
        _cnn_kernel,
        grid=(_NBLK,),
        in_specs=[
            pl.BlockSpec((_NB, 4, 7, 28), lambda i: (i, 0, 0, 0)),
            cst(140, 768), cst(1, 768),
            cst(1920, 512), cst(1, 512),
            cst(4, 256, 1024), cst(1, 1024),
            cst(1024, 16), cst(1, 16),
        ],
        out_specs=pl.BlockSpec((_NB, 16), lambda i: (i, 0)),
        out_shape=jax.ShapeDtypeStruct((2 * B, 16), jnp.float32),
    )(xq, w1, b1, w2, b2, fw1, fb1, fw2, fb2)
    return out[:, :NUM_CLASSES]


def kernel(a_imgs, b_imgs, y, conv1_w, conv1_b, conv2_w, conv2_b, fc1_w, fc1_b, fc2_w, fc2_b):
    imgs = jnp.concatenate([a_imgs, b_imgs], axis=0)
    distrs = _mnist_net_pallas(imgs, conv1_w, conv1_b, conv2_w, conv2_b,
                               fc1_w, fc1_b, fc2_w, fc2_b)
    a_distrs, b_distrs = distrs[:B], distrs[B:]
    return _sample_loss(a_distrs, b_distrs, y)
